# baseline (speedup 1.0000x reference)
import functools

import jax
import jax.numpy as jnp
from jax import lax
from jax.experimental import pallas as pl
from jax.experimental.pallas import tpu as pltpu
from jax.experimental.pallas import tpu_sc as plsc

VOCAB = 1000000
ACT_NUM = 1000
EMB = 64
B = 16384

NC = 2
NS = 16
L = 16
NW = NC * NS
BPW = B // NW
NQ = EMB // L
G = BPW // L

_mesh = plsc.VectorSubcoreMesh(core_axis_name="c", subcore_axis_name="s")


@functools.partial(
    pl.kernel,
    out_type=jax.ShapeDtypeStruct((B,), jnp.float32),
    mesh=_mesh,
    scratch_types=[
        pltpu.VMEM((BPW,), jnp.int32),
        pltpu.VMEM((BPW,), jnp.int32),
        pltpu.VMEM((BPW,), jnp.int32),
        pltpu.VMEM((BPW, EMB), jnp.float32),
        pltpu.VMEM((BPW, EMB), jnp.float32),
        pltpu.VMEM((BPW, EMB), jnp.float32),
        pltpu.VMEM((EMB,), jnp.float32),
        pltpu.VMEM((BPW,), jnp.float32),
        pltpu.VMEM((L, L), jnp.float32),
        pltpu.SemaphoreType.DMA,
    ],
    compiler_params=pltpu.CompilerParams(
        needs_layout_passes=False, use_tc_tiling_on_sc=False
    ),
)
def _transe_sc(head_hbm, tail_hbm, act_hbm, ct_hbm, at_hbm, c_hbm, out_hbm,
               hidx_v, tidx_v, aidx_v, h_v, t_v, a_v, c_v, out_v, pbuf_v, sem):
    wid = lax.axis_index("s") * NC + lax.axis_index("c")
    base = pl.multiple_of(wid * BPW, BPW)

    pltpu.sync_copy(head_hbm.at[pl.ds(base, BPW)], hidx_v)
    pltpu.sync_copy(tail_hbm.at[pl.ds(base, BPW)], tidx_v)
    pltpu.sync_copy(act_hbm.at[pl.ds(base, BPW)], aidx_v)
    pltpu.sync_copy(c_hbm, c_v)

    cp_h = pltpu.async_copy(ct_hbm.at[hidx_v], h_v, sem)
    cp_t = pltpu.async_copy(ct_hbm.at[tidx_v], t_v, sem)
    cp_a = pltpu.async_copy(at_hbm.at[aidx_v], a_v, sem)
    cp_h.wait()
    cp_t.wait()
    cp_a.wait()

    cs = [c_v[pl.ds(q * L, L)] for q in range(NQ)]
    lane = jnp.arange(L, dtype=jnp.int32)
    inv = jnp.float32(1.0 / EMB)

    def grp(g, carry):
        row0 = pl.multiple_of(g * L, L)
        for i in range(L):
            r = row0 + i
            d = None
            for q in range(NQ):
                sl = pl.ds(q * L, L)
                dq = jnp.abs(h_v[r, sl] + a_v[r, sl] - t_v[r, sl] + cs[q])
                d = dq if d is None else d + dq
            plsc.store_scatter(pbuf_v, [lane, jnp.full((L,), i, jnp.int32)], d)
        acc = pbuf_v[0, :]
        for r in range(1, L):
            acc = acc + pbuf_v[r, :]
        out_v[pl.ds(row0, L)] = acc * inv
        return carry

    lax.fori_loop(0, G, grp, 0)

    pltpu.sync_copy(out_v, out_hbm.at[pl.ds(base, BPW)])


def kernel(data, concept_table, act_table, begin, end):
    head = data[:, 0].astype(jnp.int32)
    act = data[:, 1].astype(jnp.int32)
    tail = data[:, 2].astype(jnp.int32)
    cvec = (begin - end).reshape(EMB).astype(jnp.float32)
    return _transe_sc(head, tail, act, concept_table, act_table, cvec)
